# baseline (speedup 1.0000x reference)
import jax
import jax.numpy as jnp
from jax.experimental import pallas as pl
from jax.experimental.pallas import tpu as pltpu
from jax.experimental.pallas import tpu_sc as plsc

_EMB = 128
_HID = 768
_GW = 128
_TB = 1024
_NS = 4


def _sc_gather(token_table, ids):
    bsz, seq = ids.shape
    n = bsz * seq
    spr = seq // _GW
    mesh = plsc.VectorSubcoreMesh(core_axis_name="core",
                                  subcore_axis_name="subcore")

    @pl.kernel(out_type=jax.ShapeDtypeStruct((n, _EMB), token_table.dtype),
               mesh=mesh)
    def gk(tbl_hbm, i_hbm, o_hbm):
        def body(i_vmem, o_vmem):
            pltpu.sync_copy(tbl_hbm.at[i_vmem.at[0]], o_vmem)

        pltpu.emit_pipeline(
            body,
            grid=(n // _GW,),
            in_specs=[pl.BlockSpec((1, _GW), lambda i: (i // spr, i % spr))],
            out_specs=[pl.BlockSpec((_GW, _EMB), lambda i: (i, 0))],
            core_axis_name=("core", "subcore"),
            dimension_semantics=(pltpu.PARALLEL,),
        )(i_hbm, o_hbm)

    return gk(token_table, ids)


def _tc_body(tt_ref, seg_ref, w_ref, b_ref, rw_ref, g_hbm, o_hbm,
             gbuf, obuf, gsem, osem):
    n = o_hbm.shape[0]
    nblk = n // _TB
    seq = tt_ref.shape[1]

    def gin(k, slot):
        return pltpu.make_async_copy(
            g_hbm.at[pl.ds(k * _TB, _TB), :], gbuf.at[slot], gsem.at[slot])

    def oout(k, slot):
        return pltpu.make_async_copy(
            obuf.at[slot], o_hbm.at[pl.ds(k * _TB, _TB), :], osem.at[slot])

    for k0 in range(min(_NS, nblk)):
        gin(k0, k0).start()
    seg0 = seg_ref[0:1, :]
    dseg = seg_ref[1:2, :] - seg0
    for k in range(nblk):
        slot = k % _NS
        gin(k, slot).wait()
        if k >= _NS:
            oout(k - _NS, slot).wait()
        r = (k * _TB) // seq
        c = (k * _TB) % seq
        t_row = tt_ref[pl.ds(r, 1), pl.ds(c, _TB)]
        t_col = jnp.transpose(t_row.astype(jnp.float32))
        x = gbuf[slot] + seg0 + t_col * dseg
        y = jax.lax.dot_general(
            x, w_ref[...], (((1,), (1,)), ((), ())),
            preferred_element_type=jnp.float32,
            precision=jax.lax.Precision.DEFAULT,
        ) + b_ref[...]
        var = jnp.mean(y * y, axis=-1, keepdims=True)
        obuf[slot] = y * jax.lax.rsqrt(var + 1e-6) * rw_ref[...]
        oout(k, slot).start()
        if k + _NS < nblk:
            gin(k + _NS, slot).start()
    for k in range(max(0, nblk - _NS), nblk):
        oout(k, k % _NS).wait()


def _tc_project(g, tt, seg_table, w, b, rw):
    n = g.shape[0]
    return pl.pallas_call(
        _tc_body,
        in_specs=[
            pl.BlockSpec(memory_space=pltpu.VMEM),
            pl.BlockSpec(memory_space=pltpu.VMEM),
            pl.BlockSpec(memory_space=pltpu.VMEM),
            pl.BlockSpec(memory_space=pltpu.VMEM),
            pl.BlockSpec(memory_space=pltpu.VMEM),
            pl.BlockSpec(memory_space=pl.ANY),
        ],
        out_specs=pl.BlockSpec(memory_space=pl.ANY),
        out_shape=jax.ShapeDtypeStruct((n, _HID), jnp.float32),
        scratch_shapes=[
            pltpu.VMEM((_NS, _TB, _EMB), jnp.float32),
            pltpu.VMEM((_NS, _TB, _HID), jnp.float32),
            pltpu.SemaphoreType.DMA((_NS,)),
            pltpu.SemaphoreType.DMA((_NS,)),
        ],
    )(tt, seg_table, w, b, rw, g)


def kernel(input_ids, token_type_ids, token_table, seg_table, W, b, rms_weight):
    bsz, seq = input_ids.shape
    g = _sc_gather(token_table, input_ids)
    out = _tc_project(g, token_type_ids, seg_table, W, b, rms_weight)
    return out.reshape(bsz, seq, _HID)


# speedup vs baseline: 1.0412x; 1.0412x over previous
import jax
import jax.numpy as jnp
from jax.experimental import pallas as pl
from jax.experimental.pallas import tpu as pltpu
from jax.experimental.pallas import tpu_sc as plsc

_EMB = 128
_HID = 768
_GW = 128
_TB = 2048


def _sc_gather(token_table, ids):
    bsz, seq = ids.shape
    n = bsz * seq
    spr = seq // _GW
    mesh = plsc.VectorSubcoreMesh(core_axis_name="core",
                                  subcore_axis_name="subcore")

    @pl.kernel(out_type=jax.ShapeDtypeStruct((n, _EMB), token_table.dtype),
               mesh=mesh)
    def gk(tbl_hbm, i_hbm, o_hbm):
        def body(i_vmem, o_vmem):
            pltpu.sync_copy(tbl_hbm.at[i_vmem.at[0]], o_vmem)

        pltpu.emit_pipeline(
            body,
            grid=(n // _GW,),
            in_specs=[pl.BlockSpec((1, _GW), lambda i: (i // spr, i % spr))],
            out_specs=[pl.BlockSpec((_GW, _EMB), lambda i: (i, 0))],
            core_axis_name=("core", "subcore"),
            dimension_semantics=(pltpu.PARALLEL,),
        )(i_hbm, o_hbm)

    return gk(token_table, ids)


def _tc_body(g_ref, tt_ref, seg_ref, w_ref, b_ref, rw_ref, o_ref):
    i = pl.program_id(0)
    j = pl.program_id(1)
    seg0 = seg_ref[0:1, :]
    dseg = seg_ref[1:2, :] - seg0
    t_row = tt_ref[pl.ds(i, 1), pl.ds(j * _TB, _TB)]
    t_col = jnp.transpose(t_row.astype(jnp.float32))
    x = g_ref[...] + seg0 + t_col * dseg
    y = jax.lax.dot_general(
        x, w_ref[...], (((1,), (1,)), ((), ())),
        preferred_element_type=jnp.float32,
        precision=jax.lax.Precision.DEFAULT,
    ) + b_ref[...]
    var = jnp.mean(y * y, axis=-1, keepdims=True)
    o_ref[...] = y * jax.lax.rsqrt(var + 1e-6) * rw_ref[...]


def _tc_project(g, tt, seg_table, w, b, rw):
    n = g.shape[0]
    bsz, seq = tt.shape
    bpr = seq // _TB
    return pl.pallas_call(
        _tc_body,
        grid=(bsz, bpr),
        in_specs=[
            pl.BlockSpec((_TB, _EMB), lambda i, j: (i * bpr + j, 0)),
            pl.BlockSpec((bsz, seq), lambda i, j: (0, 0)),
            pl.BlockSpec((2, _EMB), lambda i, j: (0, 0)),
            pl.BlockSpec((_HID, _EMB), lambda i, j: (0, 0)),
            pl.BlockSpec((_HID,), lambda i, j: (0,)),
            pl.BlockSpec((_HID,), lambda i, j: (0,)),
        ],
        out_specs=pl.BlockSpec((_TB, _HID), lambda i, j: (i * bpr + j, 0)),
        out_shape=jax.ShapeDtypeStruct((n, _HID), jnp.float32),
    )(g, tt, seg_table, w, b, rw)


def kernel(input_ids, token_type_ids, token_table, seg_table, W, b, rms_weight):
    bsz, seq = input_ids.shape
    g = _sc_gather(token_table, input_ids)
    out = _tc_project(g, token_type_ids, seg_table, W, b, rms_weight)
    return out.reshape(bsz, seq, _HID)
